# baseline (speedup 1.0000x reference)
import jax
import jax.numpy as jnp
from jax.experimental import pallas as pl


def kernel(pixel_values, row_weight, col_weight):
    if pixel_values.ndim != 4:
        raise ValueError('pixel_values must be a 4D tensor')
    b, h, w, _ = pixel_values.shape
    dr = row_weight.shape[1]
    dc = col_weight.shape[1]
    d = dc + dr

    row_w = row_weight[:h]
    col_w = col_weight[:w]

    hs = 2 if h % 2 == 0 else 1
    hb = h // hs

    def body(col_ref, row_ref, out_ref):
        j = pl.program_id(0) % hs
        cw = col_ref[...]
        rw = row_ref[pl.ds(j * hb, hb), :]
        out_ref[0, :, :, :dc] = jnp.broadcast_to(cw[None, :, :], (hb, w, dc))
        out_ref[0, :, :, dc:] = jnp.broadcast_to(rw[:, None, :], (hb, w, dr))

    out = pl.pallas_call(
        body,
        grid=(b * hs,),
        in_specs=[
            pl.BlockSpec((w, dc), lambda i: (0, 0)),
            pl.BlockSpec((h, dr), lambda i: (0, 0)),
        ],
        out_specs=pl.BlockSpec((1, hb, w, d), lambda i: (i // hs, i % hs, 0, 0)),
        out_shape=jax.ShapeDtypeStruct((b, h, w, d), jnp.float32),
    )(col_w, row_w)
    return out


# speedup vs baseline: 1.2258x; 1.2258x over previous
import jax
import jax.numpy as jnp
from jax.experimental import pallas as pl
from jax.experimental.pallas import tpu as pltpu


def kernel(pixel_values, row_weight, col_weight):
    if pixel_values.ndim != 4:
        raise ValueError('pixel_values must be a 4D tensor')
    b, h, w, _ = pixel_values.shape
    dr = row_weight.shape[1]
    dc = col_weight.shape[1]
    d = dc + dr

    row_w = row_weight[:h]
    col_w = col_weight[:w]

    def body(col_ref, row_ref, out_hbm, tile, sem):
        cw = col_ref[...]
        rw = row_ref[...]
        tile[:, :, :dc] = jnp.broadcast_to(cw[None, :, :], (h, w, dc))
        tile[:, :, dc:] = jnp.broadcast_to(rw[:, None, :], (h, w, dr))
        copies = [pltpu.make_async_copy(tile, out_hbm.at[ib], sem) for ib in range(b)]
        for c in copies:
            c.start()
        for c in copies:
            c.wait()

    out = pl.pallas_call(
        body,
        in_specs=[
            pl.BlockSpec(memory_space=pltpu.VMEM),
            pl.BlockSpec(memory_space=pltpu.VMEM),
        ],
        out_specs=pl.BlockSpec(memory_space=pl.ANY),
        out_shape=jax.ShapeDtypeStruct((b, h, w, d), jnp.float32),
        scratch_shapes=[
            pltpu.VMEM((h, w, d), jnp.float32),
            pltpu.SemaphoreType.DMA,
        ],
    )(col_w, row_w)
    return out
